# Initial kernel scaffold; baseline (speedup 1.0000x reference)
import jax
import jax.numpy as jnp
from jax.experimental import pallas as pl


def kernel(embs, parents, rels, rel_emb, nw_ih, nw_hh, nb_ih, nb_hh, rw_ih, rw_hh, rb_ih, rb_hh):
    raise NotImplementedError("write your pallas kernel here")


# speedup vs baseline: 8.7375x; 8.7375x over previous
import functools

import jax
import jax.numpy as jnp
from jax.experimental import pallas as pl
from jax.experimental.pallas import tpu as pltpu


def _proj_kernel(embs_ref, rels_ref, wiT_ref, rel_emb_ref, wrT_ref, bi_ref,
                 br_ref, gi_ref, grel_ref):
    Tc, B, E = embs_ref.shape
    R = rel_emb_ref.shape[0]
    x = embs_ref[...].reshape(Tc * B, E)
    gi = jnp.dot(x, wiT_ref[...], preferred_element_type=jnp.float32)
    gi_ref[...] = (gi + bi_ref[...]).reshape(Tc, B, gi.shape[-1])
    relproj = jnp.dot(rel_emb_ref[...], wrT_ref[...],
                      preferred_element_type=jnp.float32)
    r = rels_ref[...]
    onehot = (r == jax.lax.broadcasted_iota(jnp.int32, (Tc * B, R), 1)
              ).astype(jnp.float32)
    grel = jnp.dot(onehot, relproj, preferred_element_type=jnp.float32)
    grel_ref[...] = (grel + br_ref[...]).reshape(Tc, B, grel.shape[-1])


def _gates(gx, gh, b_hh, h_prev, H):
    r = jax.nn.sigmoid(gx[:, :H] + gh[:, :H] + b_hh[:, :H])
    z = jax.nn.sigmoid(gx[:, H:2 * H] + gh[:, H:2 * H] + b_hh[:, H:2 * H])
    n = jnp.tanh(gx[:, 2 * H:] + r * (gh[:, 2 * H:] + b_hh[:, 2 * H:]))
    return (1.0 - z) * n + z * h_prev


def _rec_kernel(parents_sm, gi_ref, grel_ref, whT_ref, wrT_ref, bh_ref,
                brh_ref, out_ref, red_ref, msg_ref):
    i = pl.program_id(0)
    T = pl.num_programs(0)
    t = T - 1 - i
    B, H = msg_ref.shape

    @pl.when(i == 0)
    def _init():
        red_ref[...] = jnp.zeros_like(red_ref)

    red_t = red_ref[pl.ds(t, 1), :, :][0]

    gh = jnp.dot(red_t, whT_ref[...], preferred_element_type=jnp.float32)
    h = _gates(gi_ref[0], gh, bh_ref[...], red_t, H)
    out_ref[...] = h[None]

    ghr = jnp.dot(h, wrT_ref[...], preferred_element_type=jnp.float32)
    msg_ref[...] = _gates(grel_ref[0], ghr, brh_ref[...], h, H)

    @pl.when(t > 0)
    def _scatter():
        def body(b, carry):
            p = parents_sm[t, b]
            row = msg_ref[pl.ds(b, 1), :]
            red_ref[pl.ds(p, 1), pl.ds(b, 1), :] += row[None]
            return carry

        jax.lax.fori_loop(0, B, body, 0)


def kernel(embs, parents, rels, rel_emb, nw_ih, nw_hh, nb_ih, nb_hh, rw_ih,
           rw_hh, rb_ih, rb_hh):
    B, T, E = embs.shape
    H = nw_hh.shape[1]
    G = 3 * H
    R = rel_emb.shape[0]

    embs_t = embs.astype(jnp.float32).transpose(1, 0, 2)
    rels_t = rels.astype(jnp.int32).T.reshape(T * B, 1)
    parents_t = parents.astype(jnp.int32).T
    nw_ihT = nw_ih.T
    nw_hhT = nw_hh.T
    rw_ihT = rw_ih.T
    rw_hhT = rw_hh.T
    nb_ih2 = nb_ih.reshape(1, G)
    nb_hh2 = nb_hh.reshape(1, G)
    rb_ih2 = rb_ih.reshape(1, G)
    rb_hh2 = rb_hh.reshape(1, G)

    Tc = 32
    n_chunks = T // Tc
    gi_all, grel_all = pl.pallas_call(
        _proj_kernel,
        grid=(n_chunks,),
        in_specs=[
            pl.BlockSpec((Tc, B, E), lambda i: (i, 0, 0)),
            pl.BlockSpec((Tc * B, 1), lambda i: (i, 0)),
            pl.BlockSpec((E, G), lambda i: (0, 0)),
            pl.BlockSpec((R, E), lambda i: (0, 0)),
            pl.BlockSpec((E, G), lambda i: (0, 0)),
            pl.BlockSpec((1, G), lambda i: (0, 0)),
            pl.BlockSpec((1, G), lambda i: (0, 0)),
        ],
        out_specs=[
            pl.BlockSpec((Tc, B, G), lambda i: (i, 0, 0)),
            pl.BlockSpec((Tc, B, G), lambda i: (i, 0, 0)),
        ],
        out_shape=[
            jax.ShapeDtypeStruct((T, B, G), jnp.float32),
            jax.ShapeDtypeStruct((T, B, G), jnp.float32),
        ],
    )(embs_t, rels_t, nw_ihT, rel_emb, rw_ihT, nb_ih2, rb_ih2)

    grid_spec = pltpu.PrefetchScalarGridSpec(
        num_scalar_prefetch=1,
        grid=(T,),
        in_specs=[
            pl.BlockSpec((1, B, G), lambda i, pref: (T - 1 - i, 0, 0)),
            pl.BlockSpec((1, B, G), lambda i, pref: (T - 1 - i, 0, 0)),
            pl.BlockSpec((H, G), lambda i, pref: (0, 0)),
            pl.BlockSpec((H, G), lambda i, pref: (0, 0)),
            pl.BlockSpec((1, G), lambda i, pref: (0, 0)),
            pl.BlockSpec((1, G), lambda i, pref: (0, 0)),
        ],
        out_specs=pl.BlockSpec((1, B, H), lambda i, pref: (T - 1 - i, 0, 0)),
        scratch_shapes=[
            pltpu.VMEM((T, B, H), jnp.float32),
            pltpu.VMEM((B, H), jnp.float32),
        ],
    )
    hs = pl.pallas_call(
        _rec_kernel,
        grid_spec=grid_spec,
        out_shape=jax.ShapeDtypeStruct((T, B, H), jnp.float32),
        compiler_params=pltpu.CompilerParams(
            dimension_semantics=("arbitrary",),
        ),
    )(parents_t, gi_all, grel_all, nw_hhT, rw_hhT, nb_hh2, rb_hh2)

    return hs.transpose(1, 0, 2)


# speedup vs baseline: 10.5866x; 1.2116x over previous
import functools

import jax
import jax.numpy as jnp
from jax.experimental import pallas as pl
from jax.experimental.pallas import tpu as pltpu


def _proj_kernel(embs_ref, rels_ref, wiT_ref, rel_emb_ref, wrT_ref, bi_ref,
                 br_ref, gi_ref, grel_ref):
    Tc, B, E = embs_ref.shape
    R = rel_emb_ref.shape[0]
    x = embs_ref[...].reshape(Tc * B, E).astype(jnp.bfloat16)
    gi = jnp.dot(x, wiT_ref[...], preferred_element_type=jnp.float32)
    gi_ref[...] = (gi + bi_ref[...]).reshape(Tc, B, gi.shape[-1])
    r = rels_ref[...]
    onehot = (r == jax.lax.broadcasted_iota(jnp.int32, (Tc * B, R), 1)
              ).astype(jnp.bfloat16)
    relx = jnp.dot(onehot, rel_emb_ref[...].astype(jnp.bfloat16),
                   preferred_element_type=jnp.float32)
    grel = jnp.dot(relx.astype(jnp.bfloat16), wrT_ref[...],
                   preferred_element_type=jnp.float32)
    grel_ref[...] = (grel + br_ref[...]).reshape(Tc, B, grel.shape[-1])


def _gates(gx, gh, b_hh, h_prev, H):
    r = jax.nn.sigmoid(gx[:, :H] + gh[:, :H] + b_hh[:, :H])
    z = jax.nn.sigmoid(gx[:, H:2 * H] + gh[:, H:2 * H] + b_hh[:, H:2 * H])
    n = jnp.tanh(gx[:, 2 * H:] + r * (gh[:, 2 * H:] + b_hh[:, 2 * H:]))
    return (1.0 - z) * n + z * h_prev


def _rec_kernel(parents_sm, gi_ref, grel_ref, whT_ref, wrT_ref, bh_ref,
                brh_ref, out_ref, red_ref, msg_ref):
    i = pl.program_id(0)
    T = pl.num_programs(0)
    t = T - 1 - i
    B, H = msg_ref.shape

    @pl.when(i == 0)
    def _init():
        red_ref[...] = jnp.zeros_like(red_ref)

    red_t = red_ref[pl.ds(t, 1), :, :][0]

    gh = jnp.dot(red_t.astype(jnp.bfloat16), whT_ref[...],
                 preferred_element_type=jnp.float32)
    h = _gates(gi_ref[0], gh, bh_ref[...], red_t, H)
    out_ref[...] = h[None]

    ghr = jnp.dot(h.astype(jnp.bfloat16), wrT_ref[...],
                  preferred_element_type=jnp.float32)
    msg_ref[...] = _gates(grel_ref[0], ghr, brh_ref[...], h, H)

    msg = msg_ref[...]

    @pl.when(t > 0)
    def _scatter():
        GRP = 8
        for g in range(0, B, GRP):
            ps = [parents_sm[t, b] for b in range(g, g + GRP)]
            loaded = [red_ref[pl.ds(ps[j], 1), pl.ds(g + j, 1), :]
                      for j in range(GRP)]
            for j in range(GRP):
                b = g + j
                red_ref[pl.ds(ps[j], 1), pl.ds(b, 1), :] = (
                    loaded[j] + msg[b:b + 1, :][None])


def kernel(embs, parents, rels, rel_emb, nw_ih, nw_hh, nb_ih, nb_hh, rw_ih,
           rw_hh, rb_ih, rb_hh):
    B, T, E = embs.shape
    H = nw_hh.shape[1]
    G = 3 * H
    R = rel_emb.shape[0]

    embs_t = embs.astype(jnp.float32).transpose(1, 0, 2)
    rels_t = rels.astype(jnp.int32).T.reshape(T * B, 1)
    parents_t = parents.astype(jnp.int32).T
    nw_ihT = nw_ih.T.astype(jnp.bfloat16)
    nw_hhT = nw_hh.T.astype(jnp.bfloat16)
    rw_ihT = rw_ih.T.astype(jnp.bfloat16)
    rw_hhT = rw_hh.T.astype(jnp.bfloat16)
    nb_ih2 = nb_ih.reshape(1, G)
    nb_hh2 = nb_hh.reshape(1, G)
    rb_ih2 = rb_ih.reshape(1, G)
    rb_hh2 = rb_hh.reshape(1, G)

    Tc = 32
    n_chunks = T // Tc
    gi_all, grel_all = pl.pallas_call(
        _proj_kernel,
        grid=(n_chunks,),
        in_specs=[
            pl.BlockSpec((Tc, B, E), lambda i: (i, 0, 0)),
            pl.BlockSpec((Tc * B, 1), lambda i: (i, 0)),
            pl.BlockSpec((E, G), lambda i: (0, 0)),
            pl.BlockSpec((R, E), lambda i: (0, 0)),
            pl.BlockSpec((E, G), lambda i: (0, 0)),
            pl.BlockSpec((1, G), lambda i: (0, 0)),
            pl.BlockSpec((1, G), lambda i: (0, 0)),
        ],
        out_specs=[
            pl.BlockSpec((Tc, B, G), lambda i: (i, 0, 0)),
            pl.BlockSpec((Tc, B, G), lambda i: (i, 0, 0)),
        ],
        out_shape=[
            jax.ShapeDtypeStruct((T, B, G), jnp.float32),
            jax.ShapeDtypeStruct((T, B, G), jnp.float32),
        ],
    )(embs_t, rels_t, nw_ihT, rel_emb, rw_ihT, nb_ih2, rb_ih2)

    grid_spec = pltpu.PrefetchScalarGridSpec(
        num_scalar_prefetch=1,
        grid=(T,),
        in_specs=[
            pl.BlockSpec((1, B, G), lambda i, pref: (T - 1 - i, 0, 0)),
            pl.BlockSpec((1, B, G), lambda i, pref: (T - 1 - i, 0, 0)),
            pl.BlockSpec((H, G), lambda i, pref: (0, 0)),
            pl.BlockSpec((H, G), lambda i, pref: (0, 0)),
            pl.BlockSpec((1, G), lambda i, pref: (0, 0)),
            pl.BlockSpec((1, G), lambda i, pref: (0, 0)),
        ],
        out_specs=pl.BlockSpec((1, B, H), lambda i, pref: (T - 1 - i, 0, 0)),
        scratch_shapes=[
            pltpu.VMEM((T, B, H), jnp.float32),
            pltpu.VMEM((B, H), jnp.float32),
        ],
    )
    hs = pl.pallas_call(
        _rec_kernel,
        grid_spec=grid_spec,
        out_shape=jax.ShapeDtypeStruct((T, B, H), jnp.float32),
        compiler_params=pltpu.CompilerParams(
            dimension_semantics=("arbitrary",),
        ),
    )(parents_t, gi_all, grel_all, nw_hhT, rw_hhT, nb_hh2, rb_hh2)

    return hs.transpose(1, 0, 2)


# speedup vs baseline: 11.6861x; 1.1039x over previous
import jax
import jax.numpy as jnp
from jax.experimental import pallas as pl
from jax.experimental.pallas import tpu as pltpu


def _gates(gx, gh, b_hh, h_prev, H):
    r = jax.nn.sigmoid(gx[:, :H] + gh[:, :H] + b_hh[:, :H])
    z = jax.nn.sigmoid(gx[:, H:2 * H] + gh[:, H:2 * H] + b_hh[:, H:2 * H])
    n = jnp.tanh(gx[:, 2 * H:] + r * (gh[:, 2 * H:] + b_hh[:, 2 * H:]))
    return (1.0 - z) * n + z * h_prev


def _tree_gru_kernel(parents_sm, embs_ref, rels_ref, rel_emb_ref, wiT_ref,
                     whT_ref, wrxT_ref, wrhT_ref, bi_ref, bh_ref, brx_ref,
                     brh_ref, out_ref, red_ref):
    i = pl.program_id(0)
    T = pl.num_programs(0)
    t = T - 1 - i
    B = embs_ref.shape[0]
    H = red_ref.shape[2]
    R = rel_emb_ref.shape[0]

    @pl.when(i == 0)
    def _init():
        red_ref[...] = jnp.zeros_like(red_ref)

    x = embs_ref[:, 0, 0, :].astype(jnp.bfloat16)
    gi = jnp.dot(x, wiT_ref[...], preferred_element_type=jnp.float32)
    r_ids = rels_ref[...]
    onehot = (r_ids == jax.lax.broadcasted_iota(jnp.int32, (B, R), 1)
              ).astype(jnp.bfloat16)
    relx = jnp.dot(onehot, rel_emb_ref[...], preferred_element_type=jnp.float32)
    grel = jnp.dot(relx.astype(jnp.bfloat16), wrxT_ref[...],
                   preferred_element_type=jnp.float32)

    red_t = red_ref[pl.ds(t, 1), :, :][0]
    gh = jnp.dot(red_t.astype(jnp.bfloat16), whT_ref[...],
                 preferred_element_type=jnp.float32)
    h = _gates(gi + bi_ref[...], gh, bh_ref[...], red_t, H)
    out_ref[...] = h[:, None, None, :]

    ghr = jnp.dot(h.astype(jnp.bfloat16), wrhT_ref[...],
                  preferred_element_type=jnp.float32)
    msg = _gates(grel + brx_ref[...], ghr, brh_ref[...], h, H)

    @pl.when(t > 0)
    def _scatter():
        GRP = 8
        for g in range(0, B, GRP):
            ps = [parents_sm[t, b] for b in range(g, g + GRP)]
            loaded = [red_ref[pl.ds(ps[j], 1), pl.ds(g + j, 1), :]
                      for j in range(GRP)]
            for j in range(GRP):
                b = g + j
                red_ref[pl.ds(ps[j], 1), pl.ds(b, 1), :] = (
                    loaded[j] + msg[b:b + 1, :][None])


def kernel(embs, parents, rels, rel_emb, nw_ih, nw_hh, nb_ih, nb_hh, rw_ih,
           rw_hh, rb_ih, rb_hh):
    B, T, E = embs.shape
    H = nw_hh.shape[1]
    G = 3 * H
    R = rel_emb.shape[0]

    embs4 = embs.astype(jnp.float32).reshape(B, T, 1, E)
    rels2 = rels.astype(jnp.int32).T.reshape(T * B, 1)
    parents_t = parents.astype(jnp.int32).T
    rel_emb_b = rel_emb.astype(jnp.bfloat16)
    wiT = nw_ih.T.astype(jnp.bfloat16)
    whT = nw_hh.T.astype(jnp.bfloat16)
    wrxT = rw_ih.T.astype(jnp.bfloat16)
    wrhT = rw_hh.T.astype(jnp.bfloat16)
    bi = nb_ih.reshape(1, G)
    bh = nb_hh.reshape(1, G)
    brx = rb_ih.reshape(1, G)
    brh = rb_hh.reshape(1, G)

    grid_spec = pltpu.PrefetchScalarGridSpec(
        num_scalar_prefetch=1,
        grid=(T,),
        in_specs=[
            pl.BlockSpec((B, 1, 1, E), lambda i, pref: (0, T - 1 - i, 0, 0)),
            pl.BlockSpec((B, 1), lambda i, pref: (T - 1 - i, 0)),
            pl.BlockSpec((R, E), lambda i, pref: (0, 0)),
            pl.BlockSpec((E, G), lambda i, pref: (0, 0)),
            pl.BlockSpec((H, G), lambda i, pref: (0, 0)),
            pl.BlockSpec((E, G), lambda i, pref: (0, 0)),
            pl.BlockSpec((H, G), lambda i, pref: (0, 0)),
            pl.BlockSpec((1, G), lambda i, pref: (0, 0)),
            pl.BlockSpec((1, G), lambda i, pref: (0, 0)),
            pl.BlockSpec((1, G), lambda i, pref: (0, 0)),
            pl.BlockSpec((1, G), lambda i, pref: (0, 0)),
        ],
        out_specs=pl.BlockSpec((B, 1, 1, H), lambda i, pref: (0, T - 1 - i, 0, 0)),
        scratch_shapes=[
            pltpu.VMEM((T, B, H), jnp.float32),
        ],
    )
    hs = pl.pallas_call(
        _tree_gru_kernel,
        grid_spec=grid_spec,
        out_shape=jax.ShapeDtypeStruct((B, T, 1, H), jnp.float32),
        compiler_params=pltpu.CompilerParams(
            dimension_semantics=("arbitrary",),
        ),
    )(parents_t, embs4, rels2, rel_emb_b, wiT, whT, wrxT, wrhT, bi, bh, brx,
      brh)

    return hs.reshape(B, T, H)


# speedup vs baseline: 15.0965x; 1.2918x over previous
import jax
import jax.numpy as jnp
from jax.experimental import pallas as pl
from jax.experimental.pallas import tpu as pltpu

_TC = 8


def _gates(gx, gh, b_hh, h_prev, H):
    r = jax.nn.sigmoid(gx[:, :H] + gh[:, :H] + b_hh[:, :H])
    z = jax.nn.sigmoid(gx[:, H:2 * H] + gh[:, H:2 * H] + b_hh[:, H:2 * H])
    n = jnp.tanh(gx[:, 2 * H:] + r * (gh[:, 2 * H:] + b_hh[:, 2 * H:]))
    return (1.0 - z) * n + z * h_prev


def _tree_gru_kernel(parents_sm, embs_ref, oh_ref, rel_emb_ref, wiT_ref,
                     whT_ref, wrxT_ref, wrhT_ref, bi_ref, bh_ref, brx_ref,
                     brh_ref, out_ref, red_ref):
    i = pl.program_id(0)
    n_blocks = pl.num_programs(0)
    Tc, B, E = embs_ref.shape
    H = red_ref.shape[2]

    @pl.when(i == 0)
    def _init():
        red_ref[...] = jnp.zeros_like(red_ref)

    X = embs_ref[...].reshape(Tc * B, E).astype(jnp.bfloat16)
    Gi = jnp.dot(X, wiT_ref[...],
                 preferred_element_type=jnp.float32) + bi_ref[...]
    OH = oh_ref[...].reshape(Tc * B, oh_ref.shape[2])
    relx = jnp.dot(OH, rel_emb_ref[...], preferred_element_type=jnp.float32)
    Grel = jnp.dot(relx.astype(jnp.bfloat16), wrxT_ref[...],
                   preferred_element_type=jnp.float32) + brx_ref[...]

    t_hi = (n_blocks - 1 - i) * Tc + Tc - 1

    for j in range(Tc):
        t = t_hi - j
        l = Tc - 1 - j
        red_t = red_ref[pl.ds(t, 1), :, :][0]

        gh = jnp.dot(red_t.astype(jnp.bfloat16), whT_ref[...],
                     preferred_element_type=jnp.float32)
        h = _gates(Gi[l * B:(l + 1) * B], gh, bh_ref[...], red_t, H)
        out_ref[pl.ds(l, 1)] = h[None]

        ghr = jnp.dot(h.astype(jnp.bfloat16), wrhT_ref[...],
                      preferred_element_type=jnp.float32)
        msg = _gates(Grel[l * B:(l + 1) * B], ghr, brh_ref[...], h, H)

        @pl.when(t > 0)
        def _scatter(msg=msg, t=t):
            GRP = 8
            for g in range(0, B, GRP):
                ps = [parents_sm[t, b] for b in range(g, g + GRP)]
                loaded = [red_ref[pl.ds(ps[k], 1), pl.ds(g + k, 1), :]
                          for k in range(GRP)]
                for k in range(GRP):
                    b = g + k
                    red_ref[pl.ds(ps[k], 1), pl.ds(b, 1), :] = (
                        loaded[k] + msg[b:b + 1, :][None])


def kernel(embs, parents, rels, rel_emb, nw_ih, nw_hh, nb_ih, nb_hh, rw_ih,
           rw_hh, rb_ih, rb_hh):
    B, T, E = embs.shape
    H = nw_hh.shape[1]
    G = 3 * H
    R = rel_emb.shape[0]
    Tc = _TC

    embs_t = embs.astype(jnp.float32).transpose(1, 0, 2)
    rels_t = rels.astype(jnp.int32).T
    onehot_t = (rels_t[:, :, None] == jnp.arange(R, dtype=jnp.int32)
                ).astype(jnp.bfloat16)
    parents_t = parents.astype(jnp.int32).T
    rel_emb_b = rel_emb.astype(jnp.bfloat16)
    wiT = nw_ih.T.astype(jnp.bfloat16)
    whT = nw_hh.T.astype(jnp.bfloat16)
    wrxT = rw_ih.T.astype(jnp.bfloat16)
    wrhT = rw_hh.T.astype(jnp.bfloat16)
    bi = nb_ih.reshape(1, G)
    bh = nb_hh.reshape(1, G)
    brx = rb_ih.reshape(1, G)
    brh = rb_hh.reshape(1, G)

    n_blocks = T // Tc
    grid_spec = pltpu.PrefetchScalarGridSpec(
        num_scalar_prefetch=1,
        grid=(n_blocks,),
        in_specs=[
            pl.BlockSpec((Tc, B, E), lambda i, pref: (n_blocks - 1 - i, 0, 0)),
            pl.BlockSpec((Tc, B, R), lambda i, pref: (n_blocks - 1 - i, 0, 0)),
            pl.BlockSpec((R, E), lambda i, pref: (0, 0)),
            pl.BlockSpec((E, G), lambda i, pref: (0, 0)),
            pl.BlockSpec((H, G), lambda i, pref: (0, 0)),
            pl.BlockSpec((E, G), lambda i, pref: (0, 0)),
            pl.BlockSpec((H, G), lambda i, pref: (0, 0)),
            pl.BlockSpec((1, G), lambda i, pref: (0, 0)),
            pl.BlockSpec((1, G), lambda i, pref: (0, 0)),
            pl.BlockSpec((1, G), lambda i, pref: (0, 0)),
            pl.BlockSpec((1, G), lambda i, pref: (0, 0)),
        ],
        out_specs=pl.BlockSpec((Tc, B, H),
                               lambda i, pref: (n_blocks - 1 - i, 0, 0)),
        scratch_shapes=[
            pltpu.VMEM((T, B, H), jnp.float32),
        ],
    )
    hs = pl.pallas_call(
        _tree_gru_kernel,
        grid_spec=grid_spec,
        out_shape=jax.ShapeDtypeStruct((T, B, H), jnp.float32),
        compiler_params=pltpu.CompilerParams(
            dimension_semantics=("arbitrary",),
        ),
    )(parents_t, embs_t, onehot_t, rel_emb_b, wiT, whT, wrxT, wrhT, bi, bh,
      brx, brh)

    return hs.transpose(1, 0, 2)
